# Initial kernel scaffold; baseline (speedup 1.0000x reference)
import jax
import jax.numpy as jnp
from jax.experimental import pallas as pl


def kernel(price_timestamps, event_timestamps, event_values):
    raise NotImplementedError("write your pallas kernel here")


# speedup vs baseline: 2.5217x; 2.5217x over previous
import functools

import jax
import jax.numpy as jnp
from jax.experimental import pallas as pl
from jax.experimental.pallas import tpu as pltpu

_E_TILE = 512


def _align_body(p_ref, e_ref, v_ref, out_ref, cnt_ref, *, n_events):
    Tp = p_ref.shape[1]
    D = v_ref.shape[1]
    p = p_ref[0, :]
    p_iota = jax.lax.broadcasted_iota(jnp.int32, (Tp, 1), 0)

    out_ref[...] = jnp.zeros((Tp, D), jnp.float32)
    cnt_ref[...] = jnp.zeros((1, Tp), jnp.float32)

    n_tiles = n_events // _E_TILE

    def step(t, _):
        e = e_ref[0, pl.ds(t * _E_TILE, _E_TILE)]
        dist = jnp.abs(e[:, None] - p[None, :])
        min_d = jnp.min(dist, axis=1, keepdims=True)
        idx_iota = jax.lax.broadcasted_iota(jnp.int32, dist.shape, 1)
        big = jnp.int32(Tp)
        idx = jnp.min(jnp.where(dist == min_d, idx_iota, big), axis=1)
        oh_t = (p_iota == idx[None, :]).astype(jnp.float32)
        vals = v_ref[pl.ds(t * _E_TILE, _E_TILE), :]
        out_ref[...] += jnp.dot(oh_t, vals, preferred_element_type=jnp.float32)
        cnt_ref[...] += jnp.sum(oh_t, axis=1)[None, :]
        return 0

    jax.lax.fori_loop(0, n_tiles, step, 0)
    out_ref[...] = out_ref[...] / jnp.maximum(cnt_ref[0, :], 1.0)[:, None]


def kernel(price_timestamps, event_timestamps, event_values):
    B, Tp = price_timestamps.shape
    Te = event_timestamps.shape[1]
    D = event_values.shape[2]

    out, counts = pl.pallas_call(
        functools.partial(_align_body, n_events=Te),
        grid=(B,),
        in_specs=[
            pl.BlockSpec((None, 1, Tp), lambda b: (b, 0, 0)),
            pl.BlockSpec((None, 1, Te), lambda b: (b, 0, 0)),
            pl.BlockSpec((None, Te, D), lambda b: (b, 0, 0)),
        ],
        out_specs=[
            pl.BlockSpec((None, Tp, D), lambda b: (b, 0, 0)),
            pl.BlockSpec((None, 1, Tp), lambda b: (b, 0, 0)),
        ],
        out_shape=[
            jax.ShapeDtypeStruct((B, Tp, D), jnp.float32),
            jax.ShapeDtypeStruct((B, 1, Tp), jnp.float32),
        ],
    )(
        price_timestamps.reshape(B, 1, Tp),
        event_timestamps.reshape(B, 1, Te),
        event_values,
    )
    return out, counts.reshape(B, Tp) > 0
